# baseline (speedup 1.0000x reference)
import functools

import jax
import jax.numpy as jnp
from jax import lax
from jax.experimental import pallas as pl
from jax.experimental.pallas import tpu as pltpu
from jax.experimental.pallas import tpu_sc as plsc

NC = 2
NS = 16
L = 16
PTS_PER_W = 50
SLOTS = 64
CHUNKS = SLOTS // L
BOX_F32 = PTS_PER_W * 6
ROWS_PER_MAP = 32768
C1_ROW_DELTA = 1024
MAX_ROW0 = ROWS_PER_MAP - 1 - C1_ROW_DELTA

SIZE_W = 0.1
OFFSET_W = 1.0
INV_N = 1.0 / 3200.0


def _box_comp(boxes_v, s_vec, comp):
    idx = jnp.minimum(s_vec * 6 + comp, BOX_F32 - 1)
    return plsc.load_gather(boxes_v, [idx])


def _body(sizes_hbm, offs_hbm, boxes_hbm, out_hbm,
          boxes_v, idx_v, rows_s, rows_o, accv, blk, outv, shared,
          sem, sem2):
    c = lax.axis_index("c")
    w = lax.axis_index("s")
    g = c * NS + w

    pltpu.sync_copy(boxes_hbm.at[g], boxes_v)

    lanes = []
    for k in range(CHUNKS):
        s_vec = lax.iota(jnp.int32, L) + (k * L)
        x = _box_comp(boxes_v, s_vec, 0)
        y = _box_comp(boxes_v, s_vec, 1)
        ix = (x * 128.0).astype(jnp.int32)
        iy = (y * 128.0).astype(jnp.int32)
        base = (g >> 1) * 32768 + ix * 128 + iy
        row0 = jnp.minimum(base >> 4, MAX_ROW0)
        idx_v[pl.ds(k * L, L)] = row0
        idx_v[pl.ds(SLOTS + k * L, L)] = row0 + C1_ROW_DELTA
        lanes.append(iy & (L - 1))

    cp_s = pltpu.async_copy(sizes_hbm.at[idx_v], rows_s, sem)
    cp_o = pltpu.async_copy(offs_hbm.at[idx_v], rows_o, sem2)

    tws, ths = [], []
    for k in range(CHUNKS):
        s_vec = lax.iota(jnp.int32, L) + (k * L)
        tws.append(_box_comp(boxes_v, s_vec, 2))
        ths.append(_box_comp(boxes_v, s_vec, 3))

    acc = jnp.zeros((L,), jnp.float32)
    cp_s.wait()
    for k in range(CHUNKS):
        s_vec = lax.iota(jnp.int32, L) + (k * L)
        valid = s_vec < PTS_PER_W
        ps0 = plsc.load_gather(rows_s, [s_vec, lanes[k]])
        ps1 = plsc.load_gather(rows_s, [s_vec + SLOTS, lanes[k]])
        contrib = SIZE_W * (jnp.abs(ps0 - tws[k]) + jnp.abs(ps1 - ths[k]))
        acc = acc + jnp.where(valid, contrib, 0.0)

    toxs, toys = [], []
    for k in range(CHUNKS):
        s_vec = lax.iota(jnp.int32, L) + (k * L)
        toxs.append(_box_comp(boxes_v, s_vec, 4))
        toys.append(_box_comp(boxes_v, s_vec, 5))
    cp_o.wait()
    for k in range(CHUNKS):
        s_vec = lax.iota(jnp.int32, L) + (k * L)
        valid = s_vec < PTS_PER_W
        po0 = plsc.load_gather(rows_o, [s_vec, lanes[k]])
        po1 = plsc.load_gather(rows_o, [s_vec + SLOTS, lanes[k]])
        contrib = OFFSET_W * (jnp.abs(po0 - toxs[k]) + jnp.abs(po1 - toys[k]))
        acc = acc + jnp.where(valid, contrib, 0.0)

    accv[...] = acc
    pltpu.sync_copy(accv, shared.at[w])
    plsc.subcore_barrier()

    @pl.when(w == 0)
    def _():
        pltpu.sync_copy(shared, blk)
        tot = blk[0, :]
        for r in range(1, NS):
            tot = tot + blk[r, :]
        outv[...] = jnp.full((L,), jnp.sum(tot) * INV_N, jnp.float32)
        pltpu.sync_copy(outv, out_hbm.at[c])


@functools.partial(
    pl.kernel,
    out_type=jax.ShapeDtypeStruct((NC, L), jnp.float32),
    mesh=plsc.VectorSubcoreMesh(core_axis_name="c", subcore_axis_name="s",
                                num_cores=NC, num_subcores=NS),
    compiler_params=pltpu.CompilerParams(use_tc_tiling_on_sc=False,
                                         needs_layout_passes=False,
                                         disable_bounds_checks=True,
                                         disable_semaphore_checks=True,
                                         skip_device_barrier=True),
    scratch_types=[
        pltpu.VMEM((BOX_F32,), jnp.float32),
        pltpu.VMEM((2 * SLOTS,), jnp.int32),
        pltpu.VMEM((2 * SLOTS, L), jnp.float32),
        pltpu.VMEM((2 * SLOTS, L), jnp.float32),
        pltpu.VMEM((L,), jnp.float32),
        pltpu.VMEM((NS, L), jnp.float32),
        pltpu.VMEM((L,), jnp.float32),
        pltpu.VMEM_SHARED((NS, L), jnp.float32),
        pltpu.SemaphoreType.DMA,
        pltpu.SemaphoreType.DMA,
    ],
)
def _sparse_l1_sc(*args):
    _body(*args)


@jax.jit
def kernel(predicted_sizes, predicted_offsets, target_boxes):
    sizes2d = predicted_sizes.reshape(ROWS_PER_MAP, L)
    offs2d = predicted_offsets.reshape(ROWS_PER_MAP, L)
    boxes2d = target_boxes.reshape(NC * NS, BOX_F32)
    out = _sparse_l1_sc(sizes2d, offs2d, boxes2d)
    return out[0, 0] + out[1, 0]


# speedup vs baseline: 1.2205x; 1.2205x over previous
import functools

import jax
import jax.numpy as jnp
from jax import lax
from jax.experimental import pallas as pl
from jax.experimental.pallas import tpu as pltpu
from jax.experimental.pallas import tpu_sc as plsc

NS = 16
L = 16
PTS_PER_W = 100
SLOTS = 112
CHUNKS = SLOTS // L
BOX_F32 = PTS_PER_W * 6
ROWS_PER_MAP = 32768
C1_ROW_DELTA = 1024
MAX_ROW0 = ROWS_PER_MAP - 1 - C1_ROW_DELTA

SIZE_W = 0.1
OFFSET_W = 1.0
INV_N = 1.0 / 3200.0


def _box_comp(boxes_v, s_vec, comp):
    idx = jnp.minimum(s_vec * 6 + comp, BOX_F32 - 1)
    return plsc.load_gather(boxes_v, [idx])


def _body(sizes_hbm, offs_hbm, boxes_hbm, out_hbm,
          boxes_v, idx_v, rows_s, rows_o, accv, blk, outv, shared,
          sem, sem2):
    w = lax.axis_index("s")

    pltpu.sync_copy(boxes_hbm.at[w], boxes_v)

    lanes = []
    for k in range(CHUNKS):
        s_vec = lax.iota(jnp.int32, L) + (k * L)
        x = _box_comp(boxes_v, s_vec, 0)
        y = _box_comp(boxes_v, s_vec, 1)
        ix = (x * 128.0).astype(jnp.int32)
        iy = (y * 128.0).astype(jnp.int32)
        base = w * 32768 + ix * 128 + iy
        row0 = jnp.minimum(base >> 4, MAX_ROW0)
        idx_v[pl.ds(k * L, L)] = row0
        idx_v[pl.ds(SLOTS + k * L, L)] = row0 + C1_ROW_DELTA
        lanes.append(iy & (L - 1))

    cp_s = pltpu.async_copy(sizes_hbm.at[idx_v], rows_s, sem)
    cp_o = pltpu.async_copy(offs_hbm.at[idx_v], rows_o, sem2)

    tws, ths = [], []
    for k in range(CHUNKS):
        s_vec = lax.iota(jnp.int32, L) + (k * L)
        tws.append(_box_comp(boxes_v, s_vec, 2))
        ths.append(_box_comp(boxes_v, s_vec, 3))

    acc = jnp.zeros((L,), jnp.float32)
    cp_s.wait()
    for k in range(CHUNKS):
        s_vec = lax.iota(jnp.int32, L) + (k * L)
        valid = s_vec < PTS_PER_W
        ps0 = plsc.load_gather(rows_s, [s_vec, lanes[k]])
        ps1 = plsc.load_gather(rows_s, [s_vec + SLOTS, lanes[k]])
        contrib = SIZE_W * (jnp.abs(ps0 - tws[k]) + jnp.abs(ps1 - ths[k]))
        acc = acc + jnp.where(valid, contrib, 0.0)

    toxs, toys = [], []
    for k in range(CHUNKS):
        s_vec = lax.iota(jnp.int32, L) + (k * L)
        toxs.append(_box_comp(boxes_v, s_vec, 4))
        toys.append(_box_comp(boxes_v, s_vec, 5))
    cp_o.wait()
    for k in range(CHUNKS):
        s_vec = lax.iota(jnp.int32, L) + (k * L)
        valid = s_vec < PTS_PER_W
        po0 = plsc.load_gather(rows_o, [s_vec, lanes[k]])
        po1 = plsc.load_gather(rows_o, [s_vec + SLOTS, lanes[k]])
        contrib = OFFSET_W * (jnp.abs(po0 - toxs[k]) + jnp.abs(po1 - toys[k]))
        acc = acc + jnp.where(valid, contrib, 0.0)

    accv[...] = acc
    pltpu.sync_copy(accv, shared.at[w])
    plsc.subcore_barrier()

    @pl.when(w == 0)
    def _():
        pltpu.sync_copy(shared, blk)
        tot = blk[0, :]
        for r in range(1, NS):
            tot = tot + blk[r, :]
        fin = jnp.full((L,), jnp.sum(tot) * INV_N, jnp.float32)
        lane0 = lax.iota(jnp.int32, L) == 0
        plsc.store_scatter(outv, [jnp.zeros((L,), jnp.int32)], fin, mask=lane0)
        pltpu.sync_copy(outv, out_hbm)


@functools.partial(
    pl.kernel,
    out_type=jax.ShapeDtypeStruct((1,), jnp.float32),
    mesh=plsc.VectorSubcoreMesh(core_axis_name="c", subcore_axis_name="s",
                                num_cores=1, num_subcores=NS),
    compiler_params=pltpu.CompilerParams(use_tc_tiling_on_sc=False,
                                         needs_layout_passes=False,
                                         disable_bounds_checks=True,
                                         disable_semaphore_checks=True,
                                         skip_device_barrier=True),
    scratch_types=[
        pltpu.VMEM((BOX_F32,), jnp.float32),
        pltpu.VMEM((2 * SLOTS,), jnp.int32),
        pltpu.VMEM((2 * SLOTS, L), jnp.float32),
        pltpu.VMEM((2 * SLOTS, L), jnp.float32),
        pltpu.VMEM((L,), jnp.float32),
        pltpu.VMEM((NS, L), jnp.float32),
        pltpu.VMEM((1,), jnp.float32),
        pltpu.VMEM_SHARED((NS, L), jnp.float32),
        pltpu.SemaphoreType.DMA,
        pltpu.SemaphoreType.DMA,
    ],
)
def _sparse_l1_sc(*args):
    _body(*args)


@jax.jit
def kernel(predicted_sizes, predicted_offsets, target_boxes):
    sizes2d = predicted_sizes.reshape(ROWS_PER_MAP, L)
    offs2d = predicted_offsets.reshape(ROWS_PER_MAP, L)
    boxes2d = target_boxes.reshape(NS, BOX_F32)
    out = _sparse_l1_sc(sizes2d, offs2d, boxes2d)
    return out.reshape(())
